# Initial kernel scaffold; baseline (speedup 1.0000x reference)
import jax
import jax.numpy as jnp
from jax.experimental import pallas as pl


def kernel(x, W1, b1, W2, b2, W3, b3, W4, b4, W5, b5, g1, be1, g2, be2, g3, be3, g4, be4, g5, be5, g6, be6, g7, be7, L1W, L2W, L2b, L3W, L3b):
    raise NotImplementedError("write your pallas kernel here")


# speedup vs baseline: 25.4288x; 25.4288x over previous
import jax
import jax.numpy as jnp
from jax.experimental import pallas as pl

EPS = 1e-5
K = 20
N = 1024
B = 8
NEG = -3e38
F32 = jnp.float32
HIGH = jax.lax.Precision.HIGHEST


def _graph_body(x_ref, xf_ref, m_ref):
    xr = x_ref[0]
    xc = xf_ref[0]
    sq = xr * xr
    xx_r = (sq[0:1, :] + sq[1:2, :]) + sq[2:3, :]
    sqc = xc * xc
    xx_c = (sqc[:, 0:1] + sqc[:, 1:2]) + sqc[:, 2:3]
    inner = -2.0 * jax.lax.dot_general(xc, xr, (((1,), (0,)), ((), ())),
                                       preferred_element_type=F32)
    pd = (-xx_c - inner) - xx_r
    iota_j = jax.lax.broadcasted_iota(jnp.int32, (N, N), 1)
    work = pd
    for _ in range(K):
        m = jnp.max(work, axis=1, keepdims=True)
        am = jnp.min(jnp.where(work == m, iota_j, N), axis=1, keepdims=True)
        work = jnp.where(iota_j == am, NEG, work)
    sel = work <= NEG * 0.5
    diag = iota_j == jax.lax.broadcasted_iota(jnp.int32, (N, N), 0)
    aw = jnp.where(diag | sel, 1.0, 0.0).astype(F32)
    aw = jnp.where(diag, 1.0, aw)
    deg_r = jnp.sum(aw, axis=0, keepdims=True)
    ones_c = jnp.ones((N, 1), F32)
    deg_c = jax.lax.dot_general(aw, ones_c, (((0,), (0,)), ((), ())),
                                precision=HIGH,
                                preferred_element_type=F32)
    dinv_r = jnp.where(deg_r > 0, 1.0 / jnp.sqrt(deg_r), 0.0)
    dinv_c = jnp.where(deg_c > 0, 1.0 / jnp.sqrt(deg_c), 0.0)
    m_ref[0] = aw * dinv_c * dinv_r


def _layers_body(xf_ref, m_ref, *refs):
    out_ref = refs[-1]
    mb = m_ref[0]
    h = xf_ref[0]
    inv_sqrt = 1.0 / jnp.sqrt(jnp.array(1.0 + EPS, F32))
    for li in range(5):
        w, bb, g, be = refs[4 * li:4 * li + 4]
        hw = jax.lax.dot_general(h, w[...], (((1,), (0,)), ((), ())),
                                 preferred_element_type=F32)
        agg = jax.lax.dot_general(mb, hw, (((0,), (0,)), ((), ())),
                                  precision=HIGH, preferred_element_type=F32)
        t = (agg + bb[...]) * (g[...] * inv_sqrt) + be[...]
        h = jnp.where(t >= 0, t, 0.2 * t)
    out_ref[0] = jnp.sum(h, axis=0, keepdims=True)


def _head_body(s_ref, l1_ref, g6_ref, e6_ref, l2_ref, l2b_ref,
               g7_ref, e7_ref, l3_ref, l3b_ref, out_ref):
    s = s_ref[...]
    inv_sqrt = 1.0 / jnp.sqrt(jnp.array(1.0 + EPS, F32))
    w1a = l1_ref[0:N, :]
    w1b = l1_ref[N:2 * N, :]
    y = (jax.lax.dot_general(s * (1.0 / N), w1a, (((1,), (0,)), ((), ())),
                             preferred_element_type=F32)
         + jax.lax.dot_general(s, w1b, (((1,), (0,)), ((), ())),
                               preferred_element_type=F32))
    y = y * (g6_ref[...] * inv_sqrt) + e6_ref[...]
    y = jnp.where(y >= 0, y, 0.2 * y)
    y = jax.lax.dot_general(y, l2_ref[...], (((1,), (0,)), ((), ())),
                            preferred_element_type=F32)
    y = (y + l2b_ref[...]) * (g7_ref[...] * inv_sqrt) + e7_ref[...]
    y = jnp.where(y >= 0, y, 0.2 * y)
    out_ref[...] = (jax.lax.dot_general(y, l3_ref[...], (((1,), (0,)), ((), ())),
                                        preferred_element_type=F32)
                    + l3b_ref[...])


def _full(shape):
    return pl.BlockSpec(shape, lambda b: (0,) * len(shape))


def kernel(x, W1, b1, W2, b2, W3, b3, W4, b4, W5, b5,
           g1, be1, g2, be2, g3, be3, g4, be4, g5, be5, g6, be6, g7, be7,
           L1W, L2W, L2b, L3W, L3b):
    xf = jnp.transpose(x, (0, 2, 1))
    vec = lambda v: v.reshape(1, -1)

    m = pl.pallas_call(
        _graph_body,
        grid=(B,),
        in_specs=[
            pl.BlockSpec((1, 3, N), lambda b: (b, 0, 0)),
            pl.BlockSpec((1, N, 3), lambda b: (b, 0, 0)),
        ],
        out_specs=pl.BlockSpec((1, N, N), lambda b: (b, 0, 0)),
        out_shape=jax.ShapeDtypeStruct((B, N, N), F32),
    )(x, xf)

    layer_params = []
    layer_specs = []
    for w, bb, g, be in ((W1, b1, g1, be1), (W2, b2, g2, be2),
                         (W3, b3, g3, be3), (W4, b4, g4, be4),
                         (W5, b5, g5, be5)):
        layer_params += [w, vec(bb), vec(g), vec(be)]
        layer_specs += [_full(w.shape), _full((1, bb.shape[0])),
                        _full((1, g.shape[0])), _full((1, be.shape[0]))]

    s = pl.pallas_call(
        _layers_body,
        grid=(B,),
        in_specs=[
            pl.BlockSpec((1, N, 3), lambda b: (b, 0, 0)),
            pl.BlockSpec((1, N, N), lambda b: (b, 0, 0)),
        ] + layer_specs,
        out_specs=pl.BlockSpec((1, 1, N), lambda b: (b, 0, 0)),
        out_shape=jax.ShapeDtypeStruct((B, 1, N), F32),
    )(xf, m, *layer_params)
    s = s.reshape(B, N)

    out = pl.pallas_call(
        _head_body,
        out_shape=jax.ShapeDtypeStruct((B, 40), F32),
    )(s, L1W, vec(g6), vec(be6), L2W, vec(L2b), vec(g7), vec(be7),
      L3W, vec(L3b))
    return out
